# baseline (speedup 1.0000x reference)
import jax
import jax.numpy as jnp
from jax.experimental import pallas as pl
from jax.experimental.pallas import tpu as pltpu

_SHORT_LEN = 65536
_MEDIUM_LEN = 16384
_LONG_LEN = 4096
_DIM = 512
_TOTAL = _SHORT_LEN + _MEDIUM_LEN + _LONG_LEN

_BLK = 512
_N_SHORT = _SHORT_LEN // _BLK
_N_MED = _MEDIUM_LEN // _BLK
_N_LONG = _LONG_LEN // _BLK
_N_TOT = _TOTAL // _BLK


def _assemble_body(short_ref, med_ref, long_ref, out_ref):
    i = pl.program_id(0)

    @pl.when(i < _N_SHORT)
    def _():
        out_ref[...] = short_ref[...]

    @pl.when(jnp.logical_and(i >= _N_SHORT, i < _N_SHORT + _N_MED))
    def _():
        out_ref[...] = med_ref[...]

    @pl.when(i >= _N_SHORT + _N_MED)
    def _():
        out_ref[...] = long_ref[...]


def _scatter_body(idx_ref, upd_ref, base_ref, out_ref):
    del idx_ref, base_ref
    out_ref[...] = upd_ref[...]


def _scatter_rows(assembled, updates, short_idx):
    b = updates.shape[0]
    grid_spec = pltpu.PrefetchScalarGridSpec(
        num_scalar_prefetch=1,
        grid=(b,),
        in_specs=[
            pl.BlockSpec((_DIM,), lambda i, idx: (i,)),
            pl.BlockSpec(memory_space=pl.ANY),
        ],
        out_specs=pl.BlockSpec((_DIM,), lambda i, idx: (idx[i],)),
    )
    flat = pl.pallas_call(
        _scatter_body,
        grid_spec=grid_spec,
        out_shape=jax.ShapeDtypeStruct((_TOTAL * _DIM,), jnp.float32),
        input_output_aliases={2: 0},
    )(short_idx.astype(jnp.int32), updates.reshape(b * _DIM), assembled.reshape(_TOTAL * _DIM))
    return flat.reshape(_TOTAL, _DIM)


def kernel(updates, short_idx, short_mem, medium_mem, long_mem):
    b = updates.shape[0]

    assembled = pl.pallas_call(
        _assemble_body,
        grid=(_N_TOT,),
        in_specs=[
            pl.BlockSpec((_BLK, _DIM), lambda i: (jnp.minimum(i, _N_SHORT - 1), 0)),
            pl.BlockSpec(
                (_BLK, _DIM),
                lambda i: (jnp.clip(i - _N_SHORT, 0, _N_MED - 1), 0),
            ),
            pl.BlockSpec(
                (_BLK, _DIM),
                lambda i: (jnp.clip(i - _N_SHORT - _N_MED, 0, _N_LONG - 1), 0),
            ),
        ],
        out_specs=pl.BlockSpec((_BLK, _DIM), lambda i: (i, 0)),
        out_shape=jax.ShapeDtypeStruct((_TOTAL, _DIM), jnp.float32),
    )(short_mem, medium_mem, long_mem)

    return _scatter_rows(assembled, updates, short_idx)
